# baseline (speedup 1.0000x reference)
import functools

import jax
import jax.numpy as jnp
from jax import lax
from jax.experimental import pallas as pl
from jax.experimental.pallas import tpu as pltpu
from jax.experimental.pallas import tpu_sc as plsc

N = 10000
D = 256
E = 160000
HALF = 128
BN_EPS = 1e-5

NUM_TILES = 16
CHUNK = 128
CHUNKS_PER_TILE = 80
E_PAD = NUM_TILES * CHUNKS_PER_TILE * CHUNK
NBUF = 3
NIDX = 4
PERIOD = 12
MAIN_TURNS = 72
ROWS_PER_TILE = 624
TAIL_START = NUM_TILES * ROWS_PER_TILE
TAIL_ROWS = N - TAIL_START
N_PAD = N + 8


def _sc_body(xlo, xhi, row_hbm, col_hbm, out, ibuf, d0, d1, d2,
             aggs, gs0, gs1, gs2, is0, is1, is2, is3):
    c = lax.axis_index("c")
    s = lax.axis_index("s")

    r0 = pl.multiple_of(s * ROWS_PER_TILE, 8)

    @pl.when(c == 0)
    def _():
        pltpu.sync_copy(xlo.at[pl.ds(r0, ROWS_PER_TILE)],
                        aggs.at[pl.ds(r0, ROWS_PER_TILE)])

        @pl.when(s == 0)
        def _():
            pltpu.sync_copy(xlo.at[pl.ds(TAIL_START, TAIL_ROWS)],
                            aggs.at[pl.ds(TAIL_START, TAIL_ROWS)])

    @pl.when(c == 1)
    def _():
        pltpu.sync_copy(xhi.at[pl.ds(r0, ROWS_PER_TILE)],
                        aggs.at[pl.ds(r0, ROWS_PER_TILE)])

        @pl.when(s == 0)
        def _():
            pltpu.sync_copy(xhi.at[pl.ds(TAIL_START, TAIL_ROWS)],
                            aggs.at[pl.ds(TAIL_START, TAIL_ROWS)])

    plsc.subcore_barrier()

    dbufs = (d0, d1, d2)
    gsems = (gs0, gs1, gs2)
    isems = (is0, is1, is2, is3)

    def iload(k, j):
        pltpu.async_copy(col_hbm.at[s, k], ibuf.at[j], isems[j])
        pltpu.async_copy(row_hbm.at[s, k], ibuf.at[NIDX + j], isems[j])

    def iwait(j):
        pltpu.make_async_copy(col_hbm.at[0, 0], ibuf.at[j], isems[j]).wait()
        pltpu.make_async_copy(col_hbm.at[0, 0], ibuf.at[j], isems[j]).wait()

    def gissue(k, d, j):
        @pl.when(c == 0)
        def _():
            pltpu.async_copy(xlo.at[ibuf.at[j]], dbufs[d], gsems[d])

        @pl.when(c == 1)
        def _():
            pltpu.async_copy(xhi.at[ibuf.at[j]], dbufs[d], gsems[d])

    def gwait(d):
        pltpu.make_async_copy(xlo.at[pl.ds(0, CHUNK)], dbufs[d],
                              gsems[d]).wait()

    def turn(k, d, j, tail=False):
        gwait(d)
        pltpu.sync_copy(dbufs[d], aggs.at[ibuf.at[NIDX + j]], add=True)
        pltpu.sync_copy(dbufs[d], aggs.at[ibuf.at[NIDX + j]], add=True)
        if (not tail) or (k + NIDX < CHUNKS_PER_TILE):
            iload(k + NIDX, j)
        if (not tail) or (k + NBUF < CHUNKS_PER_TILE):
            iwait((j + NBUF) % NIDX)
            gissue(k + NBUF, d, (j + NBUF) % NIDX)

    for j in range(NIDX):
        iload(j, j)
    for d in range(NBUF):
        iwait(d)
        gissue(d, d, d)

    def block(i, carry):
        k0 = i * PERIOD
        for t in range(PERIOD):
            turn(k0 + t, t % NBUF, t % NIDX)
        return carry

    lax.fori_loop(0, MAIN_TURNS // PERIOD, block, 0)
    for k in range(MAIN_TURNS, CHUNKS_PER_TILE):
        turn(k, k % NBUF, k % NIDX, tail=True)

    plsc.subcore_barrier()
    pltpu.sync_copy(aggs.at[pl.ds(r0, ROWS_PER_TILE)],
                    out.at[c, pl.ds(r0, ROWS_PER_TILE)])

    @pl.when(s == 0)
    def _():
        pltpu.sync_copy(aggs.at[pl.ds(TAIL_START, TAIL_ROWS)],
                        out.at[c, pl.ds(TAIL_START, TAIL_ROWS)])


_sc_aggregate = functools.partial(
    pl.kernel,
    out_type=jax.ShapeDtypeStruct((2, N, HALF), jnp.float32),
    mesh=plsc.VectorSubcoreMesh(core_axis_name="c", subcore_axis_name="s"),
    scratch_types=[
        pltpu.VMEM((2 * NIDX, CHUNK), jnp.int32),
        pltpu.VMEM((CHUNK, HALF), jnp.float32),
        pltpu.VMEM((CHUNK, HALF), jnp.float32),
        pltpu.VMEM((CHUNK, HALF), jnp.float32),
        pltpu.VMEM_SHARED((N_PAD, HALF), jnp.float32),
        pltpu.SemaphoreType.DMA,
        pltpu.SemaphoreType.DMA,
        pltpu.SemaphoreType.DMA,
        pltpu.SemaphoreType.DMA,
        pltpu.SemaphoreType.DMA,
        pltpu.SemaphoreType.DMA,
        pltpu.SemaphoreType.DMA,
    ],
)(_sc_body)


def _mlp1_body(xup_ref, w1_ref, b1_ref, h_ref, st_ref):
    i = pl.program_id(0)
    h = jnp.dot(xup_ref[0], w1_ref[:HALF, :],
                preferred_element_type=jnp.float32)
    h += jnp.dot(xup_ref[1], w1_ref[HALF:, :],
                 preferred_element_type=jnp.float32)
    h += b1_ref[0]
    h_ref[...] = h

    @pl.when(i == 0)
    def _():
        st_ref[...] = jnp.zeros_like(st_ref)

    zeros = jnp.zeros((6, D), jnp.float32)
    st = jnp.concatenate(
        [jnp.sum(h, axis=0, keepdims=True),
         jnp.sum(h * h, axis=0, keepdims=True),
         zeros], axis=0)
    st_ref[...] += st


def _mlp2_body(h_ref, sc_ref, sh_ref, w2_ref, b2_ref, o_ref):
    hb = jnp.maximum(h_ref[...] * sc_ref[0] + sh_ref[0], 0.0)
    o_ref[...] = jnp.dot(hb, w2_ref[...],
                         preferred_element_type=jnp.float32) + b2_ref[0]


def kernel(x, edge_index, W1, b1, gamma, beta, W2, b2):
    x_lo = x[:, :HALF]
    x_hi = x[:, HALF:]
    row = edge_index[0]
    col = edge_index[1]
    pad = E_PAD - E
    row3 = jnp.concatenate(
        [row, jnp.full((pad,), N, dtype=jnp.int32)]).reshape(
            NUM_TILES, CHUNKS_PER_TILE, CHUNK)
    col3 = jnp.concatenate(
        [col, jnp.zeros((pad,), dtype=jnp.int32)]).reshape(
            NUM_TILES, CHUNKS_PER_TILE, CHUNK)

    xup = _sc_aggregate(x_lo, x_hi, row3, col3)

    nb = 10
    blk = N // nb
    h, stats = pl.pallas_call(
        _mlp1_body,
        grid=(nb,),
        in_specs=[
            pl.BlockSpec((2, blk, HALF), lambda i: (0, i, 0)),
            pl.BlockSpec((D, D), lambda i: (0, 0)),
            pl.BlockSpec((1, D), lambda i: (0, 0)),
        ],
        out_specs=[
            pl.BlockSpec((blk, D), lambda i: (i, 0)),
            pl.BlockSpec((8, D), lambda i: (0, 0)),
        ],
        out_shape=[
            jax.ShapeDtypeStruct((N, D), jnp.float32),
            jax.ShapeDtypeStruct((8, D), jnp.float32),
        ],
    )(xup, W1, b1.reshape(1, D))

    mu = stats[0] / N
    var = stats[1] / N - mu * mu
    scale = gamma / jnp.sqrt(var + BN_EPS)
    shift = beta - mu * scale

    out = pl.pallas_call(
        _mlp2_body,
        grid=(nb,),
        in_specs=[
            pl.BlockSpec((blk, D), lambda i: (i, 0)),
            pl.BlockSpec((1, D), lambda i: (0, 0)),
            pl.BlockSpec((1, D), lambda i: (0, 0)),
            pl.BlockSpec((D, D), lambda i: (0, 0)),
            pl.BlockSpec((1, D), lambda i: (0, 0)),
        ],
        out_specs=pl.BlockSpec((blk, D), lambda i: (i, 0)),
        out_shape=jax.ShapeDtypeStruct((N, D), jnp.float32),
    )(h, scale.reshape(1, D), shift.reshape(1, D), W2, b2.reshape(1, D))

    return out


# speedup vs baseline: 1.0813x; 1.0813x over previous
import functools

import jax
import jax.numpy as jnp
from jax import lax
from jax.experimental import pallas as pl
from jax.experimental.pallas import tpu as pltpu
from jax.experimental.pallas import tpu_sc as plsc

N = 10000
D = 256
E = 160000
HALF = 128
BN_EPS = 1e-5

NUM_TILES = 16
CHUNK = 128
CHUNKS_PER_TILE = 80
E_PAD = NUM_TILES * CHUNKS_PER_TILE * CHUNK
NBUF = 3
NIDX = 4
PERIOD = 12
MAIN_TURNS = 72
ROWS_PER_TILE = 624
TAIL_START = NUM_TILES * ROWS_PER_TILE
TAIL_ROWS = N - TAIL_START
N_PAD = N + 8


def _sc_body(xlo, xhi, row_hbm, col_hbm, out, ibuf, d0, d1, d2,
             aggs, gs0, gs1, gs2, is0, is1, is2, is3):
    c = lax.axis_index("c")
    s = lax.axis_index("s")

    r0 = pl.multiple_of(s * ROWS_PER_TILE, 8)

    @pl.when(c == 0)
    def _():
        pltpu.sync_copy(xlo.at[pl.ds(r0, ROWS_PER_TILE)],
                        aggs.at[pl.ds(r0, ROWS_PER_TILE)])

        @pl.when(s == 0)
        def _():
            pltpu.sync_copy(xlo.at[pl.ds(TAIL_START, TAIL_ROWS)],
                            aggs.at[pl.ds(TAIL_START, TAIL_ROWS)])

    @pl.when(c == 1)
    def _():
        pltpu.sync_copy(xhi.at[pl.ds(r0, ROWS_PER_TILE)],
                        aggs.at[pl.ds(r0, ROWS_PER_TILE)])

        @pl.when(s == 0)
        def _():
            pltpu.sync_copy(xhi.at[pl.ds(TAIL_START, TAIL_ROWS)],
                            aggs.at[pl.ds(TAIL_START, TAIL_ROWS)])

    plsc.subcore_barrier()

    dbufs = (d0, d1, d2)
    gsems = (gs0, gs1, gs2)
    isems = (is0, is1, is2, is3)

    def iload(k, j):
        pltpu.async_copy(col_hbm.at[s, k], ibuf.at[j], isems[j])
        pltpu.async_copy(row_hbm.at[s, k], ibuf.at[NIDX + j], isems[j])

    def iwait(j):
        pltpu.make_async_copy(col_hbm.at[0, 0], ibuf.at[j], isems[j]).wait()
        pltpu.make_async_copy(col_hbm.at[0, 0], ibuf.at[j], isems[j]).wait()

    def gissue(k, d, j):
        @pl.when(c == 0)
        def _():
            pltpu.async_copy(xlo.at[ibuf.at[j]], dbufs[d], gsems[d])

        @pl.when(c == 1)
        def _():
            pltpu.async_copy(xhi.at[ibuf.at[j]], dbufs[d], gsems[d])

    def gwait(d):
        pltpu.make_async_copy(xlo.at[pl.ds(0, CHUNK)], dbufs[d],
                              gsems[d]).wait()

    def turn(k, d, j, tail=False):
        gwait(d)
        pltpu.sync_copy(dbufs[d], aggs.at[ibuf.at[NIDX + j]], add=True)
        if (not tail) or (k + NIDX < CHUNKS_PER_TILE):
            iload(k + NIDX, j)
        if (not tail) or (k + NBUF < CHUNKS_PER_TILE):
            iwait((j + NBUF) % NIDX)
            gissue(k + NBUF, d, (j + NBUF) % NIDX)

    for j in range(NIDX):
        iload(j, j)
    for d in range(NBUF):
        iwait(d)
        gissue(d, d, d)

    def block(i, carry):
        k0 = i * PERIOD
        for t in range(PERIOD):
            turn(k0 + t, t % NBUF, t % NIDX)
        return carry

    lax.fori_loop(0, MAIN_TURNS // PERIOD, block, 0)
    for k in range(MAIN_TURNS, CHUNKS_PER_TILE):
        turn(k, k % NBUF, k % NIDX, tail=True)

    plsc.subcore_barrier()
    pltpu.sync_copy(aggs.at[pl.ds(r0, ROWS_PER_TILE)],
                    out.at[c, pl.ds(r0, ROWS_PER_TILE)])

    @pl.when(s == 0)
    def _():
        pltpu.sync_copy(aggs.at[pl.ds(TAIL_START, TAIL_ROWS)],
                        out.at[c, pl.ds(TAIL_START, TAIL_ROWS)])


_sc_aggregate = functools.partial(
    pl.kernel,
    out_type=jax.ShapeDtypeStruct((2, N, HALF), jnp.float32),
    mesh=plsc.VectorSubcoreMesh(core_axis_name="c", subcore_axis_name="s"),
    scratch_types=[
        pltpu.VMEM((2 * NIDX, CHUNK), jnp.int32),
        pltpu.VMEM((CHUNK, HALF), jnp.float32),
        pltpu.VMEM((CHUNK, HALF), jnp.float32),
        pltpu.VMEM((CHUNK, HALF), jnp.float32),
        pltpu.VMEM_SHARED((N_PAD, HALF), jnp.float32),
        pltpu.SemaphoreType.DMA,
        pltpu.SemaphoreType.DMA,
        pltpu.SemaphoreType.DMA,
        pltpu.SemaphoreType.DMA,
        pltpu.SemaphoreType.DMA,
        pltpu.SemaphoreType.DMA,
        pltpu.SemaphoreType.DMA,
    ],
)(_sc_body)


def _mlp1_body(xup_ref, w1_ref, b1_ref, h_ref, st_ref):
    i = pl.program_id(0)
    h = jnp.dot(xup_ref[0], w1_ref[:HALF, :],
                preferred_element_type=jnp.float32)
    h += jnp.dot(xup_ref[1], w1_ref[HALF:, :],
                 preferred_element_type=jnp.float32)
    h += b1_ref[0]
    h_ref[...] = h

    @pl.when(i == 0)
    def _():
        st_ref[...] = jnp.zeros_like(st_ref)

    zeros = jnp.zeros((6, D), jnp.float32)
    st = jnp.concatenate(
        [jnp.sum(h, axis=0, keepdims=True),
         jnp.sum(h * h, axis=0, keepdims=True),
         zeros], axis=0)
    st_ref[...] += st


def _mlp2_body(h_ref, sc_ref, sh_ref, w2_ref, b2_ref, o_ref):
    hb = jnp.maximum(h_ref[...] * sc_ref[0] + sh_ref[0], 0.0)
    o_ref[...] = jnp.dot(hb, w2_ref[...],
                         preferred_element_type=jnp.float32) + b2_ref[0]


def kernel(x, edge_index, W1, b1, gamma, beta, W2, b2):
    x_lo = x[:, :HALF]
    x_hi = x[:, HALF:]
    row = edge_index[0]
    col = edge_index[1]
    pad = E_PAD - E
    row3 = jnp.concatenate(
        [row, jnp.full((pad,), N, dtype=jnp.int32)]).reshape(
            NUM_TILES, CHUNKS_PER_TILE, CHUNK)
    col3 = jnp.concatenate(
        [col, jnp.zeros((pad,), dtype=jnp.int32)]).reshape(
            NUM_TILES, CHUNKS_PER_TILE, CHUNK)

    xup = _sc_aggregate(x_lo, x_hi, row3, col3)

    nb = 10
    blk = N // nb
    h, stats = pl.pallas_call(
        _mlp1_body,
        grid=(nb,),
        in_specs=[
            pl.BlockSpec((2, blk, HALF), lambda i: (0, i, 0)),
            pl.BlockSpec((D, D), lambda i: (0, 0)),
            pl.BlockSpec((1, D), lambda i: (0, 0)),
        ],
        out_specs=[
            pl.BlockSpec((blk, D), lambda i: (i, 0)),
            pl.BlockSpec((8, D), lambda i: (0, 0)),
        ],
        out_shape=[
            jax.ShapeDtypeStruct((N, D), jnp.float32),
            jax.ShapeDtypeStruct((8, D), jnp.float32),
        ],
    )(xup, W1, b1.reshape(1, D))

    mu = stats[0] / N
    var = stats[1] / N - mu * mu
    scale = gamma / jnp.sqrt(var + BN_EPS)
    shift = beta - mu * scale

    out = pl.pallas_call(
        _mlp2_body,
        grid=(nb,),
        in_specs=[
            pl.BlockSpec((blk, D), lambda i: (i, 0)),
            pl.BlockSpec((1, D), lambda i: (0, 0)),
            pl.BlockSpec((1, D), lambda i: (0, 0)),
            pl.BlockSpec((D, D), lambda i: (0, 0)),
            pl.BlockSpec((1, D), lambda i: (0, 0)),
        ],
        out_specs=pl.BlockSpec((blk, D), lambda i: (i, 0)),
        out_shape=jax.ShapeDtypeStruct((N, D), jnp.float32),
    )(h, scale.reshape(1, D), shift.reshape(1, D), W2, b2.reshape(1, D))

    return out


# speedup vs baseline: 1.1869x; 1.0976x over previous
import functools

import jax
import jax.numpy as jnp
from jax import lax
from jax.experimental import pallas as pl
from jax.experimental.pallas import tpu as pltpu
from jax.experimental.pallas import tpu_sc as plsc

N = 10000
D = 256
E = 160000
HALF = 128
BN_EPS = 1e-5

NUM_TILES = 16
CHUNK = 128
CHUNKS_PER_TILE = 80
E_PAD = NUM_TILES * CHUNKS_PER_TILE * CHUNK
NBUF = 3
NIDX = 4
PERIOD = 12
MAIN_TURNS = 72
ROWS_PER_TILE = 624
TAIL_START = NUM_TILES * ROWS_PER_TILE
TAIL_ROWS = N - TAIL_START
N_PAD = N + 8


def _sc_body(xlo, xhi, row_hbm, col_hbm, out, ibuf, d0, d1, d2,
             aggs, gs0, gs1, gs2, is0, is1, is2, is3):
    c = lax.axis_index("c")
    s = lax.axis_index("s")

    r0 = pl.multiple_of(s * ROWS_PER_TILE, 8)

    @pl.when(c == 0)
    def _():
        pltpu.sync_copy(xlo.at[pl.ds(r0, ROWS_PER_TILE)],
                        aggs.at[pl.ds(r0, ROWS_PER_TILE)])

        @pl.when(s == 0)
        def _():
            pltpu.sync_copy(xlo.at[pl.ds(TAIL_START, TAIL_ROWS)],
                            aggs.at[pl.ds(TAIL_START, TAIL_ROWS)])

    @pl.when(c == 1)
    def _():
        pltpu.sync_copy(xhi.at[pl.ds(r0, ROWS_PER_TILE)],
                        aggs.at[pl.ds(r0, ROWS_PER_TILE)])

        @pl.when(s == 0)
        def _():
            pltpu.sync_copy(xhi.at[pl.ds(TAIL_START, TAIL_ROWS)],
                            aggs.at[pl.ds(TAIL_START, TAIL_ROWS)])

    plsc.subcore_barrier()

    dbufs = (d0, d1, d2)
    gsems = (gs0, gs1, gs2)
    isems = (is0, is1, is2, is3)

    def iload(k, j):
        pltpu.async_copy(col_hbm.at[s, k], ibuf.at[j], isems[j])
        pltpu.async_copy(row_hbm.at[s, k], ibuf.at[NIDX + j], isems[j])

    def iwait(j):
        pltpu.make_async_copy(col_hbm.at[0, 0], ibuf.at[j], isems[j]).wait()
        pltpu.make_async_copy(col_hbm.at[0, 0], ibuf.at[j], isems[j]).wait()

    def gissue(k, d, j):
        @pl.when(c == 0)
        def _():
            pltpu.async_copy(xlo.at[ibuf.at[j]], dbufs[d], gsems[d])

        @pl.when(c == 1)
        def _():
            pltpu.async_copy(xhi.at[ibuf.at[j]], dbufs[d], gsems[d])

    def gwait(d):
        pltpu.make_async_copy(xlo.at[pl.ds(0, CHUNK)], dbufs[d],
                              gsems[d]).wait()

    def turn(k, d, j, tail=False):
        gwait(d)
        pltpu.sync_copy(dbufs[d], aggs.at[ibuf.at[NIDX + j]], add=True)
        if (not tail) or (k + NIDX < CHUNKS_PER_TILE):
            iload(k + NIDX, j)
        if (not tail) or (k + NBUF < CHUNKS_PER_TILE):
            iwait((j + NBUF) % NIDX)
            gissue(k + NBUF, d, (j + NBUF) % NIDX)

    for j in range(NIDX):
        iload(j, j)
    for d in range(NBUF):
        iwait(d)
        gissue(d, d, d)

    def block(i, carry):
        k0 = i * PERIOD
        for t in range(PERIOD):
            turn(k0 + t, t % NBUF, t % NIDX)
        return carry

    lax.fori_loop(0, MAIN_TURNS // PERIOD, block, 0)
    for k in range(MAIN_TURNS, CHUNKS_PER_TILE):
        turn(k, k % NBUF, k % NIDX, tail=True)

    plsc.subcore_barrier()
    pltpu.sync_copy(aggs.at[pl.ds(r0, ROWS_PER_TILE)],
                    out.at[c, pl.ds(r0, ROWS_PER_TILE)])

    @pl.when(s == 0)
    def _():
        pltpu.sync_copy(aggs.at[pl.ds(TAIL_START, TAIL_ROWS)],
                        out.at[c, pl.ds(TAIL_START, TAIL_ROWS)])


_sc_aggregate = functools.partial(
    pl.kernel,
    out_type=jax.ShapeDtypeStruct((2, N, HALF), jnp.float32),
    mesh=plsc.VectorSubcoreMesh(core_axis_name="c", subcore_axis_name="s"),
    scratch_types=[
        pltpu.VMEM((2 * NIDX, CHUNK), jnp.int32),
        pltpu.VMEM((CHUNK, HALF), jnp.float32),
        pltpu.VMEM((CHUNK, HALF), jnp.float32),
        pltpu.VMEM((CHUNK, HALF), jnp.float32),
        pltpu.VMEM_SHARED((N_PAD, HALF), jnp.float32),
        pltpu.SemaphoreType.DMA,
        pltpu.SemaphoreType.DMA,
        pltpu.SemaphoreType.DMA,
        pltpu.SemaphoreType.DMA,
        pltpu.SemaphoreType.DMA,
        pltpu.SemaphoreType.DMA,
        pltpu.SemaphoreType.DMA,
    ],
)(_sc_body)


def _mlp1_body(xup_ref, w1_ref, b1_ref, h_ref, st_ref):
    i = pl.program_id(0)
    h = jnp.dot(xup_ref[0], w1_ref[:HALF, :],
                preferred_element_type=jnp.float32)
    h += jnp.dot(xup_ref[1], w1_ref[HALF:, :],
                 preferred_element_type=jnp.float32)
    h += b1_ref[0]
    h_ref[...] = h

    @pl.when(i == 0)
    def _():
        st_ref[...] = jnp.zeros_like(st_ref)

    zeros = jnp.zeros((6, D), jnp.float32)
    st = jnp.concatenate(
        [jnp.sum(h, axis=0, keepdims=True),
         jnp.sum(h * h, axis=0, keepdims=True),
         zeros], axis=0)
    st_ref[...] += st


def _mlp2_body(h_ref, st_ref, g_ref, be_ref, w2_ref, b2_ref, o_ref):
    mu = st_ref[0] * (1.0 / N)
    var = st_ref[1] * (1.0 / N) - mu * mu
    scale = g_ref[0] * jax.lax.rsqrt(var + BN_EPS)
    shift = be_ref[0] - mu * scale
    hb = jnp.maximum(h_ref[...] * scale + shift, 0.0)
    o_ref[...] = jnp.dot(hb, w2_ref[...],
                         preferred_element_type=jnp.float32) + b2_ref[0]


def kernel(x, edge_index, W1, b1, gamma, beta, W2, b2):
    x_lo = x[:, :HALF]
    x_hi = x[:, HALF:]
    row = edge_index[0]
    col = edge_index[1]
    pad = E_PAD - E
    row3 = jnp.concatenate(
        [row, jnp.full((pad,), N, dtype=jnp.int32)]).reshape(
            NUM_TILES, CHUNKS_PER_TILE, CHUNK)
    col3 = jnp.concatenate(
        [col, jnp.zeros((pad,), dtype=jnp.int32)]).reshape(
            NUM_TILES, CHUNKS_PER_TILE, CHUNK)

    xup = _sc_aggregate(x_lo, x_hi, row3, col3)

    nb = 10
    blk = N // nb
    h, stats = pl.pallas_call(
        _mlp1_body,
        grid=(nb,),
        in_specs=[
            pl.BlockSpec((2, blk, HALF), lambda i: (0, i, 0)),
            pl.BlockSpec((D, D), lambda i: (0, 0)),
            pl.BlockSpec((1, D), lambda i: (0, 0)),
        ],
        out_specs=[
            pl.BlockSpec((blk, D), lambda i: (i, 0)),
            pl.BlockSpec((8, D), lambda i: (0, 0)),
        ],
        out_shape=[
            jax.ShapeDtypeStruct((N, D), jnp.float32),
            jax.ShapeDtypeStruct((8, D), jnp.float32),
        ],
    )(xup, W1, b1.reshape(1, D))

    out = pl.pallas_call(
        _mlp2_body,
        grid=(nb,),
        in_specs=[
            pl.BlockSpec((blk, D), lambda i: (i, 0)),
            pl.BlockSpec((8, D), lambda i: (0, 0)),
            pl.BlockSpec((1, D), lambda i: (0, 0)),
            pl.BlockSpec((1, D), lambda i: (0, 0)),
            pl.BlockSpec((D, D), lambda i: (0, 0)),
            pl.BlockSpec((1, D), lambda i: (0, 0)),
        ],
        out_specs=pl.BlockSpec((blk, D), lambda i: (i, 0)),
        out_shape=jax.ShapeDtypeStruct((N, D), jnp.float32),
    )(h, stats, gamma.reshape(1, D), beta.reshape(1, D), W2,
      b2.reshape(1, D))

    return out


# speedup vs baseline: 1.1946x; 1.0065x over previous
import functools

import jax
import jax.numpy as jnp
from jax import lax
from jax.experimental import pallas as pl
from jax.experimental.pallas import tpu as pltpu
from jax.experimental.pallas import tpu_sc as plsc

N = 10000
D = 256
E = 160000
HALF = 128
BN_EPS = 1e-5

NUM_TILES = 16
CHUNK = 128
CHUNKS_PER_TILE = 80
E_PAD = NUM_TILES * CHUNKS_PER_TILE * CHUNK
NBUF = 3
NIDX = 4
PERIOD = 12
MAIN_TURNS = 72
ROWS_PER_TILE = 624
TAIL_START = NUM_TILES * ROWS_PER_TILE
TAIL_ROWS = N - TAIL_START
N_PAD = N + 8


def _sc_body(xlo, xhi, row_hbm, col_hbm, out, ibuf, d0, d1, d2,
             aggs, gs0, gs1, gs2, is0, is1, is2, is3):
    c = lax.axis_index("c")
    s = lax.axis_index("s")

    r0 = pl.multiple_of(s * ROWS_PER_TILE, 8)

    @pl.when(c == 0)
    def _():
        pltpu.sync_copy(xlo.at[pl.ds(r0, ROWS_PER_TILE)],
                        aggs.at[pl.ds(r0, ROWS_PER_TILE)])

        @pl.when(s == 0)
        def _():
            pltpu.sync_copy(xlo.at[pl.ds(TAIL_START, TAIL_ROWS)],
                            aggs.at[pl.ds(TAIL_START, TAIL_ROWS)])

    @pl.when(c == 1)
    def _():
        pltpu.sync_copy(xhi.at[pl.ds(r0, ROWS_PER_TILE)],
                        aggs.at[pl.ds(r0, ROWS_PER_TILE)])

        @pl.when(s == 0)
        def _():
            pltpu.sync_copy(xhi.at[pl.ds(TAIL_START, TAIL_ROWS)],
                            aggs.at[pl.ds(TAIL_START, TAIL_ROWS)])

    plsc.subcore_barrier()

    dbufs = (d0, d1, d2)
    gsems = (gs0, gs1, gs2)
    isems = (is0, is1, is2, is3)

    def iload(k, j):
        pltpu.async_copy(col_hbm.at[s, k], ibuf.at[j], isems[j])
        pltpu.async_copy(row_hbm.at[s, k], ibuf.at[NIDX + j], isems[j])

    def iwait(j):
        pltpu.make_async_copy(col_hbm.at[0, 0], ibuf.at[j], isems[j]).wait()
        pltpu.make_async_copy(col_hbm.at[0, 0], ibuf.at[j], isems[j]).wait()

    def gissue(k, d, j):
        @pl.when(c == 0)
        def _():
            pltpu.async_copy(xlo.at[ibuf.at[j]], dbufs[d], gsems[d])

        @pl.when(c == 1)
        def _():
            pltpu.async_copy(xhi.at[ibuf.at[j]], dbufs[d], gsems[d])

    def gwait(d):
        pltpu.make_async_copy(xlo.at[pl.ds(0, CHUNK)], dbufs[d],
                              gsems[d]).wait()

    def turn(k, d, j, tail=False):
        gwait(d)
        pltpu.sync_copy(dbufs[d], aggs.at[ibuf.at[NIDX + j]], add=True)
        if (not tail) or (k + NIDX < CHUNKS_PER_TILE):
            iload(k + NIDX, j)
        if (not tail) or (k + NBUF < CHUNKS_PER_TILE):
            iwait((j + NBUF) % NIDX)
            gissue(k + NBUF, d, (j + NBUF) % NIDX)

    for j in range(NIDX):
        iload(j, j)
    for d in range(NBUF):
        iwait(d)
        gissue(d, d, d)

    def block(i, carry):
        k0 = i * PERIOD
        for t in range(PERIOD):
            turn(k0 + t, t % NBUF, t % NIDX)
        return carry

    lax.fori_loop(0, MAIN_TURNS // PERIOD, block, 0)
    for k in range(MAIN_TURNS, CHUNKS_PER_TILE):
        turn(k, k % NBUF, k % NIDX, tail=True)

    plsc.subcore_barrier()
    pltpu.sync_copy(aggs.at[pl.ds(r0, ROWS_PER_TILE)],
                    out.at[c, pl.ds(r0, ROWS_PER_TILE)])

    @pl.when(s == 0)
    def _():
        pltpu.sync_copy(aggs.at[pl.ds(TAIL_START, TAIL_ROWS)],
                        out.at[c, pl.ds(TAIL_START, TAIL_ROWS)])


_sc_aggregate = functools.partial(
    pl.kernel,
    out_type=jax.ShapeDtypeStruct((2, N, HALF), jnp.float32),
    mesh=plsc.VectorSubcoreMesh(core_axis_name="c", subcore_axis_name="s"),
    scratch_types=[
        pltpu.VMEM((2 * NIDX, CHUNK), jnp.int32),
        pltpu.VMEM((CHUNK, HALF), jnp.float32),
        pltpu.VMEM((CHUNK, HALF), jnp.float32),
        pltpu.VMEM((CHUNK, HALF), jnp.float32),
        pltpu.VMEM_SHARED((N_PAD, HALF), jnp.float32),
        pltpu.SemaphoreType.DMA,
        pltpu.SemaphoreType.DMA,
        pltpu.SemaphoreType.DMA,
        pltpu.SemaphoreType.DMA,
        pltpu.SemaphoreType.DMA,
        pltpu.SemaphoreType.DMA,
        pltpu.SemaphoreType.DMA,
    ],
)(_sc_body)


def _mlp_body(xup_ref, w1_ref, b1_ref, g_ref, be_ref, w2_ref, b2_ref,
              o_ref, h_scr, st_scr):
    p = pl.program_id(0)
    i = pl.program_id(1)
    blk = h_scr.shape[0] // pl.num_programs(1)

    @pl.when(p == 0)
    def _():
        h = jnp.dot(xup_ref[0], w1_ref[:HALF, :],
                    preferred_element_type=jnp.float32)
        h += jnp.dot(xup_ref[1], w1_ref[HALF:, :],
                     preferred_element_type=jnp.float32)
        h += b1_ref[0]
        h_scr[pl.ds(i * blk, blk), :] = h

        @pl.when(i == 0)
        def _():
            st_scr[...] = jnp.zeros_like(st_scr)

        zeros = jnp.zeros((6, D), jnp.float32)
        st = jnp.concatenate(
            [jnp.sum(h, axis=0, keepdims=True),
             jnp.sum(h * h, axis=0, keepdims=True),
             zeros], axis=0)
        st_scr[...] += st

    @pl.when(p == 1)
    def _():
        mu = st_scr[0] * (1.0 / N)
        var = st_scr[1] * (1.0 / N) - mu * mu
        scale = g_ref[0] * jax.lax.rsqrt(var + BN_EPS)
        shift = be_ref[0] - mu * scale
        hb = jnp.maximum(h_scr[pl.ds(i * blk, blk), :] * scale + shift,
                         0.0)
        o_ref[...] = jnp.dot(hb, w2_ref[...],
                             preferred_element_type=jnp.float32) + b2_ref[0]


def kernel(x, edge_index, W1, b1, gamma, beta, W2, b2):
    x_lo = x[:, :HALF]
    x_hi = x[:, HALF:]
    row = edge_index[0]
    col = edge_index[1]
    pad = E_PAD - E
    row3 = jnp.concatenate(
        [row, jnp.full((pad,), N, dtype=jnp.int32)]).reshape(
            NUM_TILES, CHUNKS_PER_TILE, CHUNK)
    col3 = jnp.concatenate(
        [col, jnp.zeros((pad,), dtype=jnp.int32)]).reshape(
            NUM_TILES, CHUNKS_PER_TILE, CHUNK)

    xup = _sc_aggregate(x_lo, x_hi, row3, col3)

    nb = 10
    blk = N // nb
    out = pl.pallas_call(
        _mlp_body,
        grid=(2, nb),
        in_specs=[
            pl.BlockSpec((2, blk, HALF), lambda p, i: (0, i * (1 - p), 0)),
            pl.BlockSpec((D, D), lambda p, i: (0, 0)),
            pl.BlockSpec((1, D), lambda p, i: (0, 0)),
            pl.BlockSpec((1, D), lambda p, i: (0, 0)),
            pl.BlockSpec((1, D), lambda p, i: (0, 0)),
            pl.BlockSpec((D, D), lambda p, i: (0, 0)),
            pl.BlockSpec((1, D), lambda p, i: (0, 0)),
        ],
        out_specs=pl.BlockSpec((blk, D), lambda p, i: (i * p, 0)),
        out_shape=jax.ShapeDtypeStruct((N, D), jnp.float32),
        scratch_shapes=[
            pltpu.VMEM((N, D), jnp.float32),
            pltpu.VMEM((8, D), jnp.float32),
        ],
    )(xup, W1, b1.reshape(1, D), gamma.reshape(1, D), beta.reshape(1, D),
      W2, b2.reshape(1, D))

    return out
